# baseline (device time: 193154 ns/iter reference)
import jax
import jax.numpy as jnp
from jax import lax
from jax.experimental import pallas as pl
from jax.experimental.pallas import tpu as pltpu

N_DEV = 32
SUB = 2


def kernel(x, w_mat):
    m_tot, k_shard = x.shape
    _, n = w_mat.shape
    m_per = m_tot // N_DEV
    nh = n // 2
    ns = nh // SUB

    def body(x_ref, w_ref, out_ref, wbf_ref, p_ref, cw_ref, ccw_ref,
             cw_send, cw_recv, ccw_send, ccw_recv):
        my = lax.axis_index("i")
        left = lax.rem(my - 1 + N_DEV, N_DEV)
        right = lax.rem(my + 1, N_DEV)

        barrier_sem = pltpu.get_barrier_semaphore()
        for nbr in (left, right):
            pl.semaphore_signal(
                barrier_sem, inc=1,
                device_id=(nbr,), device_id_type=pl.DeviceIdType.MESH,
            )
        pl.semaphore_wait(barrier_sem, 2)

        wbf_ref[...] = w_ref[...].astype(jnp.bfloat16)
        p_ref[...] = jnp.dot(
            x_ref[...].astype(jnp.bfloat16), wbf_ref[...],
            preferred_element_type=jnp.float32,
        ).astype(jnp.bfloat16)

        def partial_for(c, col0):
            return p_ref[pl.ds(c * m_per, m_per), col0:col0 + nh]

        def mk(dir_ref, send_sems, recv_sems, dst_dev, h, s):
            src_slot = N_DEV - 1 if h == 0 else h - 1
            return pltpu.make_async_remote_copy(
                src_ref=dir_ref.at[src_slot, :, s * ns:(s + 1) * ns],
                dst_ref=dir_ref.at[h, :, s * ns:(s + 1) * ns],
                send_sem=send_sems.at[h, s],
                recv_sem=recv_sems.at[h, s],
                device_id=(dst_dev,),
                device_id_type=pl.DeviceIdType.MESH,
            )

        cw_ref[N_DEV - 1] = partial_for(lax.rem(my - 1 + N_DEV, N_DEV), 0)
        ccw_ref[N_DEV - 1] = partial_for(lax.rem(my + 1, N_DEV), nh)
        for s in range(SUB):
            mk(cw_ref, cw_send, cw_recv, right, 0, s).start()
            mk(ccw_ref, ccw_send, ccw_recv, left, 0, s).start()

        for h in range(N_DEV - 1):
            c_cw = lax.rem(my - h - 2 + 2 * N_DEV, N_DEV)
            c_ccw = lax.rem(my + h + 2, N_DEV)
            p_cw = partial_for(c_cw, 0).astype(jnp.float32)
            p_ccw = partial_for(c_ccw, nh).astype(jnp.float32)

            for s in range(SUB):
                sl = slice(s * ns, (s + 1) * ns)
                for dir_ref, send_sems, recv_sems, dst, p, col0 in (
                    (cw_ref, cw_send, cw_recv, right, p_cw, 0),
                    (ccw_ref, ccw_send, ccw_recv, left, p_ccw, nh),
                ):
                    mk(dir_ref, send_sems, recv_sems, dst, h, s).wait_recv()
                    acc = dir_ref[h, :, sl].astype(jnp.float32) + p[:, sl]
                    if h < N_DEV - 2:
                        dir_ref[h, :, sl] = acc.astype(jnp.bfloat16)
                        mk(dir_ref, send_sems, recv_sems, dst, h + 1, s).start()
                    else:
                        out_ref[:, col0 + s * ns:col0 + (s + 1) * ns] = (
                            acc * jax.nn.sigmoid(acc))

        for h in range(N_DEV - 1):
            for s in range(SUB):
                mk(cw_ref, cw_send, cw_recv, right, h, s).wait_send()
                mk(ccw_ref, ccw_send, ccw_recv, left, h, s).wait_send()

    return pl.pallas_call(
        body,
        out_shape=jax.ShapeDtypeStruct((m_per, n), jnp.float32),
        in_specs=[
            pl.BlockSpec(memory_space=pltpu.VMEM),
            pl.BlockSpec(memory_space=pltpu.VMEM),
        ],
        out_specs=pl.BlockSpec(memory_space=pltpu.VMEM),
        scratch_shapes=[
            pltpu.VMEM((k_shard, n), jnp.bfloat16),
            pltpu.VMEM((m_tot, n), jnp.bfloat16),
            pltpu.VMEM((N_DEV, m_per, nh), jnp.bfloat16),
            pltpu.VMEM((N_DEV, m_per, nh), jnp.bfloat16),
            pltpu.SemaphoreType.DMA((N_DEV - 1, SUB)),
            pltpu.SemaphoreType.DMA((N_DEV - 1, SUB)),
            pltpu.SemaphoreType.DMA((N_DEV - 1, SUB)),
            pltpu.SemaphoreType.DMA((N_DEV - 1, SUB)),
        ],
        compiler_params=pltpu.CompilerParams(
            collective_id=0, vmem_limit_bytes=100 * 1024 * 1024,
        ),
    )(x, w_mat)


# device time: 116188 ns/iter; 1.6624x vs baseline; 1.6624x over previous
import jax
import jax.numpy as jnp
from jax import lax
from jax.experimental import pallas as pl
from jax.experimental.pallas import tpu as pltpu

N_DEV = 32
SUB = 2

_MESH_COORDS = [
    (x, y, z)
    for z in range(4)
    for y in range(4)
    for x in ((0, 1) if y % 2 == 0 else (1, 0))
]
_LOGICAL_OF = {c: k for k, c in enumerate(_MESH_COORDS)}

_HAM_YZ = [
    (0, 0), (1, 0), (2, 0), (3, 0),
    (3, 1), (2, 1), (1, 1), (0, 1),
    (0, 2), (1, 2), (2, 2), (3, 2),
    (3, 3), (2, 3), (1, 3), (0, 3),
]
_RING_COORDS = [(0, y, z) for (y, z) in _HAM_YZ] + [
    (1, y, z) for (y, z) in reversed(_HAM_YZ)
]
RING_LOGICAL = [_LOGICAL_OF[c] for c in _RING_COORDS]
RPOS_OF = [0] * N_DEV
for _r, _m in enumerate(RING_LOGICAL):
    RPOS_OF[_m] = _r


def kernel(x, w_mat):
    m_tot, k_shard = x.shape
    _, n = w_mat.shape
    m_per = m_tot // N_DEV
    nh = n // 2
    ns = nh // SUB

    def body(x_ref, w_ref, ring_l_ref, rpos_ref, out_ref, wbf_ref, p_ref,
             cw_ref, ccw_ref, cw_send, cw_recv, ccw_send, ccw_recv):
        my = lax.axis_index("i")

        idx = lax.broadcasted_iota(jnp.int32, (1, N_DEV), 1)
        ring_l = ring_l_ref[...]
        rpos_t = rpos_ref[...]

        def lut(tbl, i):
            return jnp.sum(jnp.where(idx == i, tbl, 0))

        r = lut(rpos_t, my)
        right = lut(ring_l, lax.rem(r + 1, N_DEV))
        left = lut(ring_l, lax.rem(r - 1 + N_DEV, N_DEV))

        barrier_sem = pltpu.get_barrier_semaphore()
        for nbr in (left, right):
            pl.semaphore_signal(
                barrier_sem, inc=1,
                device_id=(nbr,), device_id_type=pl.DeviceIdType.MESH,
            )
        pl.semaphore_wait(barrier_sem, 2)

        wbf_ref[...] = w_ref[...].astype(jnp.bfloat16)
        p_ref[...] = jnp.dot(
            x_ref[...].astype(jnp.bfloat16), wbf_ref[...],
            preferred_element_type=jnp.float32,
        ).astype(jnp.bfloat16)

        def partial_for(c, col0):
            return p_ref[pl.ds(c * m_per, m_per), col0:col0 + nh]

        def mk(dir_ref, send_sems, recv_sems, dst_dev, h, s):
            src_slot = N_DEV - 1 if h == 0 else h - 1
            return pltpu.make_async_remote_copy(
                src_ref=dir_ref.at[src_slot, :, s * ns:(s + 1) * ns],
                dst_ref=dir_ref.at[h, :, s * ns:(s + 1) * ns],
                send_sem=send_sems.at[h, s],
                recv_sem=recv_sems.at[h, s],
                device_id=(dst_dev,),
                device_id_type=pl.DeviceIdType.MESH,
            )

        cw_ref[N_DEV - 1] = partial_for(left, 0)
        ccw_ref[N_DEV - 1] = partial_for(right, nh)
        for s in range(SUB):
            mk(cw_ref, cw_send, cw_recv, right, 0, s).start()
            mk(ccw_ref, ccw_send, ccw_recv, left, 0, s).start()

        for h in range(N_DEV - 1):
            c_cw = lut(ring_l, lax.rem(r - h - 2 + 2 * N_DEV, N_DEV))
            c_ccw = lut(ring_l, lax.rem(r + h + 2, N_DEV))
            p_cw = partial_for(c_cw, 0).astype(jnp.float32)
            p_ccw = partial_for(c_ccw, nh).astype(jnp.float32)

            for s in range(SUB):
                sl = slice(s * ns, (s + 1) * ns)
                for dir_ref, send_sems, recv_sems, dst, p, col0 in (
                    (cw_ref, cw_send, cw_recv, right, p_cw, 0),
                    (ccw_ref, ccw_send, ccw_recv, left, p_ccw, nh),
                ):
                    mk(dir_ref, send_sems, recv_sems, dst, h, s).wait_recv()
                    acc = dir_ref[h, :, sl].astype(jnp.float32) + p[:, sl]
                    if h < N_DEV - 2:
                        dir_ref[h, :, sl] = acc.astype(jnp.bfloat16)
                        mk(dir_ref, send_sems, recv_sems, dst, h + 1, s).start()
                    else:
                        out_ref[:, col0 + s * ns:col0 + (s + 1) * ns] = (
                            acc * jax.nn.sigmoid(acc))

        for h in range(N_DEV - 1):
            for s in range(SUB):
                mk(cw_ref, cw_send, cw_recv, right, h, s).wait_send()
                mk(ccw_ref, ccw_send, ccw_recv, left, h, s).wait_send()

    return pl.pallas_call(
        body,
        out_shape=jax.ShapeDtypeStruct((m_per, n), jnp.float32),
        in_specs=[
            pl.BlockSpec(memory_space=pltpu.VMEM),
            pl.BlockSpec(memory_space=pltpu.VMEM),
            pl.BlockSpec(memory_space=pltpu.VMEM),
            pl.BlockSpec(memory_space=pltpu.VMEM),
        ],
        out_specs=pl.BlockSpec(memory_space=pltpu.VMEM),
        scratch_shapes=[
            pltpu.VMEM((k_shard, n), jnp.bfloat16),
            pltpu.VMEM((m_tot, n), jnp.bfloat16),
            pltpu.VMEM((N_DEV, m_per, nh), jnp.bfloat16),
            pltpu.VMEM((N_DEV, m_per, nh), jnp.bfloat16),
            pltpu.SemaphoreType.DMA((N_DEV - 1, SUB)),
            pltpu.SemaphoreType.DMA((N_DEV - 1, SUB)),
            pltpu.SemaphoreType.DMA((N_DEV - 1, SUB)),
            pltpu.SemaphoreType.DMA((N_DEV - 1, SUB)),
        ],
        compiler_params=pltpu.CompilerParams(
            collective_id=0, vmem_limit_bytes=100 * 1024 * 1024,
        ),
    )(x, w_mat,
      jnp.array([RING_LOGICAL], dtype=jnp.int32),
      jnp.array([RPOS_OF], dtype=jnp.int32))


# device time: 107356 ns/iter; 1.7992x vs baseline; 1.0823x over previous
import jax
import jax.numpy as jnp
from jax import lax
from jax.experimental import pallas as pl
from jax.experimental.pallas import tpu as pltpu

N_DEV = 32
SUB = 4

_MESH_COORDS = [
    (x, y, z)
    for z in range(4)
    for y in range(4)
    for x in ((0, 1) if y % 2 == 0 else (1, 0))
]
_LOGICAL_OF = {c: k for k, c in enumerate(_MESH_COORDS)}

_HAM_YZ = [
    (0, 0), (1, 0), (2, 0), (3, 0),
    (3, 1), (2, 1), (1, 1), (0, 1),
    (0, 2), (1, 2), (2, 2), (3, 2),
    (3, 3), (2, 3), (1, 3), (0, 3),
]
_RING_COORDS = [(0, y, z) for (y, z) in _HAM_YZ] + [
    (1, y, z) for (y, z) in reversed(_HAM_YZ)
]
RING_LOGICAL = [_LOGICAL_OF[c] for c in _RING_COORDS]
RPOS_OF = [0] * N_DEV
for _r, _m in enumerate(RING_LOGICAL):
    RPOS_OF[_m] = _r


def kernel(x, w_mat):
    m_tot, k_shard = x.shape
    _, n = w_mat.shape
    m_per = m_tot // N_DEV
    nh = n // 2
    ns = nh // SUB

    def body(x_ref, w_ref, ring_l_ref, rpos_ref, out_ref, wbf_ref, p_ref,
             cw_ref, ccw_ref, cw_send, cw_recv, ccw_send, ccw_recv):
        my = lax.axis_index("i")

        idx = lax.broadcasted_iota(jnp.int32, (1, N_DEV), 1)
        ring_l = ring_l_ref[...]
        rpos_t = rpos_ref[...]

        def lut(tbl, i):
            return jnp.sum(jnp.where(idx == i, tbl, 0))

        r = lut(rpos_t, my)
        right = lut(ring_l, lax.rem(r + 1, N_DEV))
        left = lut(ring_l, lax.rem(r - 1 + N_DEV, N_DEV))

        barrier_sem = pltpu.get_barrier_semaphore()
        for nbr in (left, right):
            pl.semaphore_signal(
                barrier_sem, inc=1,
                device_id=(nbr,), device_id_type=pl.DeviceIdType.MESH,
            )
        pl.semaphore_wait(barrier_sem, 2)

        wbf_ref[...] = w_ref[...].astype(jnp.bfloat16)
        p_ref[...] = jnp.dot(
            x_ref[...].astype(jnp.bfloat16), wbf_ref[...],
            preferred_element_type=jnp.float32,
        ).astype(jnp.bfloat16)

        def partial_for(c, col0):
            return p_ref[pl.ds(c * m_per, m_per), col0:col0 + nh]

        def mk(dir_ref, send_sems, recv_sems, dst_dev, h, s):
            src_slot = N_DEV - 1 if h == 0 else h - 1
            return pltpu.make_async_remote_copy(
                src_ref=dir_ref.at[src_slot, :, s * ns:(s + 1) * ns],
                dst_ref=dir_ref.at[h, :, s * ns:(s + 1) * ns],
                send_sem=send_sems.at[h, s],
                recv_sem=recv_sems.at[h, s],
                device_id=(dst_dev,),
                device_id_type=pl.DeviceIdType.MESH,
            )

        cw_ref[N_DEV - 1] = partial_for(left, 0)
        ccw_ref[N_DEV - 1] = partial_for(right, nh)
        for s in range(SUB):
            mk(cw_ref, cw_send, cw_recv, right, 0, s).start()
            mk(ccw_ref, ccw_send, ccw_recv, left, 0, s).start()

        for h in range(N_DEV - 1):
            c_cw = lut(ring_l, lax.rem(r - h - 2 + 2 * N_DEV, N_DEV))
            c_ccw = lut(ring_l, lax.rem(r + h + 2, N_DEV))
            p_cw = partial_for(c_cw, 0).astype(jnp.float32)
            p_ccw = partial_for(c_ccw, nh).astype(jnp.float32)

            for s in range(SUB):
                sl = slice(s * ns, (s + 1) * ns)
                for dir_ref, send_sems, recv_sems, dst, p, col0 in (
                    (cw_ref, cw_send, cw_recv, right, p_cw, 0),
                    (ccw_ref, ccw_send, ccw_recv, left, p_ccw, nh),
                ):
                    mk(dir_ref, send_sems, recv_sems, dst, h, s).wait_recv()
                    acc = dir_ref[h, :, sl].astype(jnp.float32) + p[:, sl]
                    if h < N_DEV - 2:
                        dir_ref[h, :, sl] = acc.astype(jnp.bfloat16)
                        mk(dir_ref, send_sems, recv_sems, dst, h + 1, s).start()
                    else:
                        out_ref[:, col0 + s * ns:col0 + (s + 1) * ns] = (
                            acc * jax.nn.sigmoid(acc))

        for h in range(N_DEV - 1):
            for s in range(SUB):
                mk(cw_ref, cw_send, cw_recv, right, h, s).wait_send()
                mk(ccw_ref, ccw_send, ccw_recv, left, h, s).wait_send()

    return pl.pallas_call(
        body,
        out_shape=jax.ShapeDtypeStruct((m_per, n), jnp.float32),
        in_specs=[
            pl.BlockSpec(memory_space=pltpu.VMEM),
            pl.BlockSpec(memory_space=pltpu.VMEM),
            pl.BlockSpec(memory_space=pltpu.VMEM),
            pl.BlockSpec(memory_space=pltpu.VMEM),
        ],
        out_specs=pl.BlockSpec(memory_space=pltpu.VMEM),
        scratch_shapes=[
            pltpu.VMEM((k_shard, n), jnp.bfloat16),
            pltpu.VMEM((m_tot, n), jnp.bfloat16),
            pltpu.VMEM((N_DEV, m_per, nh), jnp.bfloat16),
            pltpu.VMEM((N_DEV, m_per, nh), jnp.bfloat16),
            pltpu.SemaphoreType.DMA((N_DEV - 1, SUB)),
            pltpu.SemaphoreType.DMA((N_DEV - 1, SUB)),
            pltpu.SemaphoreType.DMA((N_DEV - 1, SUB)),
            pltpu.SemaphoreType.DMA((N_DEV - 1, SUB)),
        ],
        compiler_params=pltpu.CompilerParams(
            collective_id=0, vmem_limit_bytes=100 * 1024 * 1024,
        ),
    )(x, w_mat,
      jnp.array([RING_LOGICAL], dtype=jnp.int32),
      jnp.array([RPOS_OF], dtype=jnp.int32))


# device time: 107340 ns/iter; 1.7995x vs baseline; 1.0001x over previous
import jax
import jax.numpy as jnp
from jax import lax
from jax.experimental import pallas as pl
from jax.experimental.pallas import tpu as pltpu

N_DEV = 32
SUB = 4

_MESH_COORDS = [
    (x, y, z)
    for z in range(4)
    for y in range(4)
    for x in ((0, 1) if y % 2 == 0 else (1, 0))
]
_LOGICAL_OF = {c: k for k, c in enumerate(_MESH_COORDS)}

_HAM_YZ = [
    (0, 0), (1, 0), (2, 0), (3, 0),
    (3, 1), (2, 1), (1, 1), (0, 1),
    (0, 2), (1, 2), (2, 2), (3, 2),
    (3, 3), (2, 3), (1, 3), (0, 3),
]
_RING_COORDS = [(0, y, z) for (y, z) in _HAM_YZ] + [
    (1, y, z) for (y, z) in reversed(_HAM_YZ)
]
RING_LOGICAL = [_LOGICAL_OF[c] for c in _RING_COORDS]
RPOS_OF = [0] * N_DEV
for _r, _m in enumerate(RING_LOGICAL):
    RPOS_OF[_m] = _r


def kernel(x, w_mat):
    m_tot, k_shard = x.shape
    _, n = w_mat.shape
    m_per = m_tot // N_DEV
    nh = n // 2
    ns = nh // SUB

    def body(x_ref, w_ref, ring_l_ref, rpos_ref, out_ref, wbf_ref, p_ref,
             cw_ref, ccw_ref, cw_send, cw_recv, ccw_send, ccw_recv):
        my = lax.axis_index("i")

        idx = lax.broadcasted_iota(jnp.int32, (1, N_DEV), 1)
        ring_l = ring_l_ref[...]
        rpos_t = rpos_ref[...]

        def lut(tbl, i):
            return jnp.sum(jnp.where(idx == i, tbl, 0))

        r = lut(rpos_t, my)
        right = lut(ring_l, lax.rem(r + 1, N_DEV))
        left = lut(ring_l, lax.rem(r - 1 + N_DEV, N_DEV))

        barrier_sem = pltpu.get_barrier_semaphore()
        for nbr in (left, right):
            pl.semaphore_signal(
                barrier_sem, inc=1,
                device_id=(nbr,), device_id_type=pl.DeviceIdType.MESH,
            )
        pl.semaphore_wait(barrier_sem, 2)

        wbf_ref[...] = w_ref[...].astype(jnp.bfloat16)
        p_ref[...] = jnp.dot(
            x_ref[...].astype(jnp.bfloat16), wbf_ref[...],
            preferred_element_type=jnp.float32,
        ).astype(jnp.bfloat16)

        def partial_for(c, col0):
            return p_ref[pl.ds(c * m_per, m_per), col0:col0 + nh]

        def mk(dir_ref, send_sems, recv_sems, dst_dev, h, s):
            src_slot = N_DEV - 1 if h == 0 else h - 1
            return pltpu.make_async_remote_copy(
                src_ref=dir_ref.at[src_slot, :, s * ns:(s + 1) * ns],
                dst_ref=dir_ref.at[h, :, s * ns:(s + 1) * ns],
                send_sem=send_sems.at[h, s],
                recv_sem=recv_sems.at[h, s],
                device_id=(dst_dev,),
                device_id_type=pl.DeviceIdType.MESH,
            )

        cw_ref[N_DEV - 1] = partial_for(left, 0)
        ccw_ref[N_DEV - 1] = partial_for(right, nh)
        for s in range(SUB):
            mk(cw_ref, cw_send, cw_recv, right, 0, s).start()
            mk(ccw_ref, ccw_send, ccw_recv, left, 0, s).start()

        for h in range(N_DEV - 1):
            c_cw = lut(ring_l, lax.rem(r - h - 2 + 2 * N_DEV, N_DEV))
            c_ccw = lut(ring_l, lax.rem(r + h + 2, N_DEV))
            p_cw = partial_for(c_cw, 0)
            p_ccw = partial_for(c_ccw, nh)

            for s in range(SUB):
                sl = slice(s * ns, (s + 1) * ns)
                for dir_ref, send_sems, recv_sems, dst, p, col0 in (
                    (cw_ref, cw_send, cw_recv, right, p_cw, 0),
                    (ccw_ref, ccw_send, ccw_recv, left, p_ccw, nh),
                ):
                    mk(dir_ref, send_sems, recv_sems, dst, h, s).wait_recv()
                    if h < N_DEV - 2:
                        dir_ref[h, :, sl] = dir_ref[h, :, sl] + p[:, sl]
                        mk(dir_ref, send_sems, recv_sems, dst, h + 1, s).start()
                    else:
                        acc = (dir_ref[h, :, sl].astype(jnp.float32)
                               + p[:, sl].astype(jnp.float32))
                        out_ref[:, col0 + s * ns:col0 + (s + 1) * ns] = (
                            acc * jax.nn.sigmoid(acc))

        for h in range(N_DEV - 1):
            for s in range(SUB):
                mk(cw_ref, cw_send, cw_recv, right, h, s).wait_send()
                mk(ccw_ref, ccw_send, ccw_recv, left, h, s).wait_send()

    return pl.pallas_call(
        body,
        out_shape=jax.ShapeDtypeStruct((m_per, n), jnp.float32),
        in_specs=[
            pl.BlockSpec(memory_space=pltpu.VMEM),
            pl.BlockSpec(memory_space=pltpu.VMEM),
            pl.BlockSpec(memory_space=pltpu.VMEM),
            pl.BlockSpec(memory_space=pltpu.VMEM),
        ],
        out_specs=pl.BlockSpec(memory_space=pltpu.VMEM),
        scratch_shapes=[
            pltpu.VMEM((k_shard, n), jnp.bfloat16),
            pltpu.VMEM((m_tot, n), jnp.bfloat16),
            pltpu.VMEM((N_DEV, m_per, nh), jnp.bfloat16),
            pltpu.VMEM((N_DEV, m_per, nh), jnp.bfloat16),
            pltpu.SemaphoreType.DMA((N_DEV - 1, SUB)),
            pltpu.SemaphoreType.DMA((N_DEV - 1, SUB)),
            pltpu.SemaphoreType.DMA((N_DEV - 1, SUB)),
            pltpu.SemaphoreType.DMA((N_DEV - 1, SUB)),
        ],
        compiler_params=pltpu.CompilerParams(
            collective_id=0, vmem_limit_bytes=100 * 1024 * 1024,
        ),
    )(x, w_mat,
      jnp.array([RING_LOGICAL], dtype=jnp.int32),
      jnp.array([RPOS_OF], dtype=jnp.int32))


# device time: 105002 ns/iter; 1.8395x vs baseline; 1.0223x over previous
import jax
import jax.numpy as jnp
from jax import lax
from jax.experimental import pallas as pl
from jax.experimental.pallas import tpu as pltpu

N_DEV = 32
SUB = 4

_MESH_COORDS = [
    (x, y, z)
    for z in range(4)
    for y in range(4)
    for x in ((0, 1) if y % 2 == 0 else (1, 0))
]
_LOGICAL_OF = {c: k for k, c in enumerate(_MESH_COORDS)}

_HAM_YZ = [
    (0, 0), (1, 0), (2, 0), (3, 0),
    (3, 1), (2, 1), (1, 1), (0, 1),
    (0, 2), (1, 2), (2, 2), (3, 2),
    (3, 3), (2, 3), (1, 3), (0, 3),
]
_RING_COORDS = [(0, y, z) for (y, z) in _HAM_YZ] + [
    (1, y, z) for (y, z) in reversed(_HAM_YZ)
]
RING_LOGICAL = [_LOGICAL_OF[c] for c in _RING_COORDS]
RPOS_OF = [0] * N_DEV
for _r, _m in enumerate(RING_LOGICAL):
    RPOS_OF[_m] = _r


def kernel(x, w_mat):
    m_tot, k_shard = x.shape
    _, n = w_mat.shape
    m_per = m_tot // N_DEV
    nh = n // 2
    ns = nh // SUB

    def body(x_ref, w_ref, ring_l_ref, rpos_ref, out_ref, wbf_ref, xr_ref,
             pr_ref, cw_ref, ccw_ref, cw_send, cw_recv, ccw_send, ccw_recv):
        my = lax.axis_index("i")

        idx = lax.broadcasted_iota(jnp.int32, (1, N_DEV), 1)

        def lut(tbl, i):
            return jnp.sum(jnp.where(idx == i, tbl, 0))

        r = lut(rpos_ref[...], my)
        ring_l = ring_l_ref[...]
        right = lut(ring_l, lax.rem(r + 1, N_DEV))
        left = lut(ring_l, lax.rem(r - 1 + N_DEV, N_DEV))

        barrier_sem = pltpu.get_barrier_semaphore()
        for nbr in (left, right):
            pl.semaphore_signal(
                barrier_sem, inc=1,
                device_id=(nbr,), device_id_type=pl.DeviceIdType.MESH,
            )
        pl.semaphore_wait(barrier_sem, 2)

        wbf_ref[...] = w_ref[...].astype(jnp.bfloat16)
        for rho, c in enumerate(RING_LOGICAL):
            xr_ref[rho * m_per:(rho + 1) * m_per, :] = (
                x_ref[c * m_per:(c + 1) * m_per, :].astype(jnp.bfloat16))

        def prow_cw(h):
            return lax.rem(r - h - 2 + 4 * N_DEV, N_DEV) * m_per

        def prow_ccw(h):
            return lax.rem(r + h + 2, N_DEV) * m_per

        def mk(dir_ref, send_sems, recv_sems, dst_dev, h, s):
            src_slot = N_DEV - 1 if h == 0 else h - 1
            return pltpu.make_async_remote_copy(
                src_ref=dir_ref.at[src_slot, :, s * ns:(s + 1) * ns],
                dst_ref=dir_ref.at[h, :, s * ns:(s + 1) * ns],
                send_sem=send_sems.at[h, s],
                recv_sem=recv_sems.at[h, s],
                device_id=(dst_dev,),
                device_id_type=pl.DeviceIdType.MESH,
            )

        cw_ref[N_DEV - 1] = jnp.dot(
            xr_ref[pl.ds(prow_cw(-1), m_per), :], wbf_ref[:, :nh],
            preferred_element_type=jnp.float32).astype(jnp.bfloat16)
        ccw_ref[N_DEV - 1] = jnp.dot(
            xr_ref[pl.ds(prow_ccw(-1), m_per), :], wbf_ref[:, nh:],
            preferred_element_type=jnp.float32).astype(jnp.bfloat16)
        for s in range(SUB):
            mk(cw_ref, cw_send, cw_recv, right, 0, s).start()
            mk(ccw_ref, ccw_send, ccw_recv, left, 0, s).start()

        pr_ref[...] = jnp.dot(
            xr_ref[...], wbf_ref[...],
            preferred_element_type=jnp.float32).astype(jnp.bfloat16)

        for h in range(N_DEV - 1):
            p_cw = pr_ref[pl.ds(prow_cw(h), m_per), :nh]
            p_ccw = pr_ref[pl.ds(prow_ccw(h), m_per), nh:]

            for s in range(SUB):
                sl = slice(s * ns, (s + 1) * ns)
                for dir_ref, send_sems, recv_sems, dst, p, col0 in (
                    (cw_ref, cw_send, cw_recv, right, p_cw, 0),
                    (ccw_ref, ccw_send, ccw_recv, left, p_ccw, nh),
                ):
                    mk(dir_ref, send_sems, recv_sems, dst, h, s).wait_recv()
                    if h < N_DEV - 2:
                        dir_ref[h, :, sl] = dir_ref[h, :, sl] + p[:, sl]
                        mk(dir_ref, send_sems, recv_sems, dst, h + 1, s).start()
                    else:
                        acc = (dir_ref[h, :, sl].astype(jnp.float32)
                               + p[:, sl].astype(jnp.float32))
                        out_ref[:, col0 + s * ns:col0 + (s + 1) * ns] = (
                            acc * jax.nn.sigmoid(acc))

        for h in range(N_DEV - 1):
            for s in range(SUB):
                mk(cw_ref, cw_send, cw_recv, right, h, s).wait_send()
                mk(ccw_ref, ccw_send, ccw_recv, left, h, s).wait_send()

    return pl.pallas_call(
        body,
        out_shape=jax.ShapeDtypeStruct((m_per, n), jnp.float32),
        in_specs=[
            pl.BlockSpec(memory_space=pltpu.VMEM),
            pl.BlockSpec(memory_space=pltpu.VMEM),
            pl.BlockSpec(memory_space=pltpu.VMEM),
            pl.BlockSpec(memory_space=pltpu.VMEM),
        ],
        out_specs=pl.BlockSpec(memory_space=pltpu.VMEM),
        scratch_shapes=[
            pltpu.VMEM((k_shard, n), jnp.bfloat16),
            pltpu.VMEM((m_tot, k_shard), jnp.bfloat16),
            pltpu.VMEM((m_tot, n), jnp.bfloat16),
            pltpu.VMEM((N_DEV, m_per, nh), jnp.bfloat16),
            pltpu.VMEM((N_DEV, m_per, nh), jnp.bfloat16),
            pltpu.SemaphoreType.DMA((N_DEV - 1, SUB)),
            pltpu.SemaphoreType.DMA((N_DEV - 1, SUB)),
            pltpu.SemaphoreType.DMA((N_DEV - 1, SUB)),
            pltpu.SemaphoreType.DMA((N_DEV - 1, SUB)),
        ],
        compiler_params=pltpu.CompilerParams(
            collective_id=0, vmem_limit_bytes=100 * 1024 * 1024,
        ),
    )(x, w_mat,
      jnp.array([RING_LOGICAL], dtype=jnp.int32),
      jnp.array([RPOS_OF], dtype=jnp.int32))
